# Initial kernel scaffold; baseline (speedup 1.0000x reference)
import jax
import jax.numpy as jnp
from jax.experimental import pallas as pl


def kernel(A, B):
    raise NotImplementedError("write your pallas kernel here")


# speedup vs baseline: 320.4194x; 320.4194x over previous
import jax
import jax.numpy as jnp
from jax.experimental import pallas as pl

_INTERPRET = False

_N = 4096
_BLK = 256
_M = 3


def _nn_body(s_ref, d_ref, dist_ref, g_ref):
    s = s_ref[...]
    dx = d_ref[0:1, :]
    dy = d_ref[1:2, :]
    dz = d_ref[2:3, :]
    tx = s[:, 0:1] - dx
    ty = s[:, 1:2] - dy
    tz = s[:, 2:3] - dz
    d2 = tx * tx + ty * ty + tz * tz
    mind = jnp.min(d2, axis=1, keepdims=True)
    iota = jax.lax.broadcasted_iota(jnp.int32, d2.shape, 1)
    bidx = jnp.min(jnp.where(d2 <= mind, iota, _N), axis=1, keepdims=True)
    onehot = iota == bidx
    gx = jnp.sum(jnp.where(onehot, dx, 0.0), axis=1)
    gy = jnp.sum(jnp.where(onehot, dy, 0.0), axis=1)
    gz = jnp.sum(jnp.where(onehot, dz, 0.0), axis=1)
    dist_ref[0, 0, :] = jnp.sqrt(jnp.maximum(mind[:, 0], 0.0))
    g_ref[0, 0, :] = gx
    g_ref[0, 1, :] = gy
    g_ref[0, 2, :] = gz


def _nn(src_pts, dstT):
    nblk = _N // _BLK
    dist, g = pl.pallas_call(
        _nn_body,
        grid=(nblk,),
        in_specs=[
            pl.BlockSpec((_BLK, _M), lambda i: (i, 0)),
            pl.BlockSpec((_M, _N), lambda i: (0, 0)),
        ],
        out_specs=[
            pl.BlockSpec((1, 1, _BLK), lambda i: (i, 0, 0)),
            pl.BlockSpec((1, _M, _BLK), lambda i: (i, 0, 0)),
        ],
        out_shape=[
            jax.ShapeDtypeStruct((nblk, 1, _BLK), jnp.float32),
            jax.ShapeDtypeStruct((nblk, _M, _BLK), jnp.float32),
        ],
        interpret=_INTERPRET,
    )(src_pts, dstT)
    dist = dist.reshape(-1)
    G = g.transpose(0, 2, 1).reshape(-1, _M)
    return dist, G


def _fit(A, B):
    m = A.shape[1]
    cA = jnp.mean(A, axis=0)
    cB = jnp.mean(B, axis=0)
    AA = A - cA
    BB = B - cB
    H = AA.T @ BB
    U, S, Vt = jnp.linalg.svd(H, full_matrices=False)
    R = Vt.T @ U.T
    sign = jnp.where(jnp.linalg.det(R) < 0, -1.0, 1.0)
    Vt = Vt.at[m - 1, :].multiply(sign)
    R = Vt.T @ U.T
    t = cB - R @ cA
    T = jnp.eye(m + 1, dtype=A.dtype)
    T = T.at[:m, :m].set(R)
    T = T.at[:m, m].set(t)
    return T, R, t


def kernel(A, B):
    max_iterations = 20
    tolerance = 0.001
    dstT = B.T

    def cond(c):
        _, _, done, i = c
        return jnp.logical_and(i < max_iterations, jnp.logical_not(done))

    def body(c):
        src, prev_error, done, i = c
        dist, G = _nn(src, dstT)
        _, R, t = _fit(src, G)
        src_new = src @ R.T + t
        mean_error = jnp.mean(dist)
        converged = jnp.abs(prev_error - mean_error) < tolerance
        return (src_new, mean_error, done | converged, i + 1)

    init = (A, jnp.zeros((), A.dtype), jnp.array(False), jnp.array(0, jnp.int32))
    src, _, _, _ = jax.lax.while_loop(cond, body, init)
    T, _, _ = _fit(A, src)
    return T
